# Initial kernel scaffold; baseline (speedup 1.0000x reference)
import jax
import jax.numpy as jnp
from jax.experimental import pallas as pl


def kernel(x_scalar, x_spherical, coord, batch_idx, sW1, sb1, sW2, sb2, lW0, lb0, lW1, lW2, vW0, vb0, vW2):
    raise NotImplementedError("write your pallas kernel here")


# speedup vs baseline: 13.8953x; 13.8953x over previous
import functools
import math

import jax
import jax.numpy as jnp
import numpy as np
from jax import lax
from jax.experimental import pallas as pl
from jax.experimental.pallas import tpu as pltpu
from jax.experimental.pallas import tpu_sc as plsc

_NSEG = 1024
_T = 2048

_SQ3 = math.sqrt(3.0)

_SM = np.zeros((5, 16), np.float32)
_SM[0, 1] = _SM[0, 3] = 1.0
_SM[1, 5] = _SM[1, 7] = 1.0
_SM[2, 0] = _SM[2, 4] = -1.0 / _SQ3
_SM[2, 8] = 2.0 / _SQ3
_SM[3, 2] = _SM[3, 6] = 1.0
_SM[4, 0] = 1.0
_SM[4, 4] = -1.0
_ZR = np.zeros((1, 16), np.float32)
_ZR[0, [0, 4, 8]] = 1.0
_DR = np.zeros((1, 16), np.float32)
_DR[0, [0, 4, 8]] = 1.0 / _SQ3
_GS = np.repeat(np.eye(16, dtype=np.float32), 5, axis=0)


def _tc_body(n_total, xs_ref, e0_ref, e2a_ref, e2b_ref, w1_ref, b1_ref,
             wh2_ref, w2_ref, b2_ref, v_ref, gs_ref, gst_ref, mix_ref,
             out_ref):
    i = pl.program_id(0)
    f32 = jnp.float32
    a = jnp.concatenate([xs_ref[...], e0_ref[...]], axis=1)
    h = jnp.dot(a, w1_ref[...], preferred_element_type=f32) + b1_ref[...]
    lane = lax.broadcasted_iota(jnp.int32, h.shape, 1)
    ag = h * jax.nn.sigmoid(jnp.where(lane < 64, h, jnp.abs(h)))
    p = jnp.dot(ag, w2_ref[...], preferred_element_type=f32) + b2_ref[...]
    e2b = jnp.where(lane < 96, e2b_ref[...], 0.0)
    a2 = jnp.concatenate([e2a_ref[...], e2b], axis=1)
    h2 = jnp.dot(a2, wh2_ref[...], preferred_element_type=f32)
    n2sq = jnp.dot(h2 * h2, gs_ref[...], preferred_element_type=f32)
    sig2 = jax.nn.sigmoid(jnp.sqrt(n2sq))
    g2 = h2 * jnp.dot(sig2, gst_ref[...], preferred_element_type=f32)
    s2 = jnp.dot(g2, v_ref[...], preferred_element_type=f32)
    zero = p[:, 2:3] * p[:, 0:1]
    second = s2 * p[:, 1:2]
    dn = jnp.sqrt(jnp.sum(second * second, axis=1, keepdims=True))
    feat = jnp.concatenate([zero, dn, second], axis=1)
    out16 = jnp.dot(feat, mix_ref[...], preferred_element_type=f32)
    rows = i * _T + lax.broadcasted_iota(jnp.int32, out16.shape, 0)
    out_ref[...] = jnp.where(rows < n_total, out16, 0.0)


def _mat_rows(x_scalar, x_spherical, w1, b1, wh2, w2, b2, v):
    n = x_scalar.shape[0]
    g = pl.cdiv(n, _T)
    npad = g * _T
    mix = jnp.asarray(np.concatenate([_ZR, _DR, _SM], axis=0))
    return pl.pallas_call(
        functools.partial(_tc_body, n),
        grid=(g,),
        in_specs=[
            pl.BlockSpec((_T, 128), lambda i: (i, 0)),
            pl.BlockSpec((_T, 128), lambda i: (i, 0)),
            pl.BlockSpec((_T, 128), lambda i: (i, 2)),
            pl.BlockSpec((_T, 128), lambda i: (i, 3)),
            pl.BlockSpec((256, 128), lambda i: (0, 0)),
            pl.BlockSpec((1, 128), lambda i: (0, 0)),
            pl.BlockSpec((256, 80), lambda i: (0, 0)),
            pl.BlockSpec((128, 8), lambda i: (0, 0)),
            pl.BlockSpec((1, 8), lambda i: (0, 0)),
            pl.BlockSpec((80, 5), lambda i: (0, 0)),
            pl.BlockSpec((80, 16), lambda i: (0, 0)),
            pl.BlockSpec((16, 80), lambda i: (0, 0)),
            pl.BlockSpec((7, 16), lambda i: (0, 0)),
        ],
        out_specs=pl.BlockSpec((_T, 16), lambda i: (i, 0)),
        out_shape=jax.ShapeDtypeStruct((npad, 16), jnp.float32),
    )(x_scalar, x_spherical, x_spherical, x_spherical, w1, b1, wh2, w2, b2,
      v, jnp.asarray(_GS), jnp.asarray(_GS.T), mix)


_SZ = 64
_SPW = 32


def _seg_sum(mats, idx2d, karr):
    mesh = plsc.VectorSubcoreMesh(core_axis_name="c", subcore_axis_name="s")

    @functools.partial(
        pl.kernel,
        out_type=jax.ShapeDtypeStruct((_NSEG, 16), jnp.float32),
        mesh=mesh,
        scratch_types=[
            pltpu.VMEM((16,), jnp.int32),
            pltpu.VMEM((_SZ,), jnp.int32),
            pltpu.VMEM((_SZ, 16), jnp.float32),
            pltpu.VMEM((_SPW + 1, 16), jnp.float32),
        ],
    )
    def sc_kernel(mat_hbm, idx_hbm, karr_hbm, out_hbm, kb, idx_v, buf, acc):
        c = lax.axis_index("c")
        s = lax.axis_index("s")
        wid = s * 2 + c
        base_seg = wid * _SPW
        for r in range(_SPW + 1):
            acc[r, :] = jnp.zeros((16,), jnp.float32)
        pltpu.sync_copy(karr_hbm.at[pl.ds(wid * 16, 16)], kb)
        kbv = kb[...]
        k0 = kbv[0]
        k1 = kbv[1]

        def step(k, carry):
            pltpu.sync_copy(idx_hbm.at[k], idx_v)
            pltpu.sync_copy(mat_hbm.at[pl.ds(k * _SZ, _SZ)], buf)
            for g in range(_SZ // 16):
                iv = idx_v[pl.ds(g * 16, 16)] - base_seg
                for r in range(16):
                    rel = iv[r]
                    ok = (rel >= 0) & (rel < _SPW)
                    ridx = jnp.where(ok, rel, _SPW)
                    plsc.addupdate(acc.at[ridx], buf[g * 16 + r, :])
            return carry

        lax.fori_loop(k0, k1, step, 0)
        pltpu.sync_copy(acc.at[pl.ds(0, _SPW)],
                        out_hbm.at[pl.ds(base_seg, _SPW)])

    return sc_kernel(mats, idx2d, karr)


def _prep_weights(sW1, sb1, sW2, sb2, lW0, lb0, lW2, vW0, vb0, vW2):
    f32 = jnp.float32
    w1 = jnp.zeros((256, 128), f32)
    w1 = w1.at[:128, :64].set(sW1)
    w1 = w1.at[128:, 64:].set(lW0 * (1.0 / math.sqrt(128.0)))
    b1 = jnp.concatenate([sb1, lb0]).reshape(1, 128)
    wh2_core = (lW2[:, None, :, None] * jnp.eye(5, dtype=f32)[None, :, None, :])
    wh2_core = wh2_core.reshape(160, 80) * (1.0 / math.sqrt(32.0))
    wh2 = jnp.zeros((256, 80), f32).at[64:224, :].set(wh2_core)
    w2 = jnp.zeros((128, 8), f32)
    w2 = w2.at[:64, 0:2].set(sW2)
    w2 = w2.at[64:, 2].set(vW0[:, 0] * (1.0 / math.sqrt(64.0)))
    b2 = jnp.zeros((8,), f32).at[0:2].set(sb2).at[2].set(vb0[0]).reshape(1, 8)
    v = (vW2[:, 0][:, None, None] * jnp.eye(5, dtype=f32)[None, :, :])
    v = v.reshape(80, 5) * (1.0 / math.sqrt(16.0))
    return w1, b1, wh2, w2, b2, v


def kernel(x_scalar, x_spherical, coord, batch_idx, sW1, sb1, sW2, sb2,
           lW0, lb0, lW1, lW2, vW0, vb0, vW2):
    del coord, lW1
    n = x_scalar.shape[0]
    w1, b1, wh2, w2, b2, v = _prep_weights(sW1, sb1, sW2, sb2, lW0, lb0,
                                           lW2, vW0, vb0, vW2)
    mats = _mat_rows(x_scalar, x_spherical, w1, b1, wh2, w2, b2, v)
    npad = mats.shape[0]
    bip = jnp.concatenate(
        [batch_idx,
         jnp.full((npad - n,), _NSEG - 1, jnp.int32)])
    starts = jnp.searchsorted(bip, jnp.arange(0, _NSEG + 1, _SPW)
                              ).astype(jnp.int32)
    k0 = starts[:-1] // _SZ
    k1 = (starts[1:] + _SZ - 1) // _SZ
    karr = jnp.zeros((32, 16), jnp.int32)
    karr = karr.at[:, 0].set(k0).at[:, 1].set(k1).reshape(-1)
    seg = _seg_sum(mats, bip.reshape(-1, _SZ), karr)
    return seg[:, :9].reshape(_NSEG, 3, 3)


# speedup vs baseline: 14.9171x; 1.0735x over previous
import functools
import math

import jax
import jax.numpy as jnp
import numpy as np
from jax import lax
from jax.experimental import pallas as pl
from jax.experimental.pallas import tpu as pltpu
from jax.experimental.pallas import tpu_sc as plsc

_NSEG = 1024
_T = 4096

_SQ3 = math.sqrt(3.0)

_SM = np.zeros((5, 16), np.float32)
_SM[0, 1] = _SM[0, 3] = 1.0
_SM[1, 5] = _SM[1, 7] = 1.0
_SM[2, 0] = _SM[2, 4] = -1.0 / _SQ3
_SM[2, 8] = 2.0 / _SQ3
_SM[3, 2] = _SM[3, 6] = 1.0
_SM[4, 0] = 1.0
_SM[4, 4] = -1.0
_ZR = np.zeros((1, 16), np.float32)
_ZR[0, [0, 4, 8]] = 1.0
_DR = np.zeros((1, 16), np.float32)
_DR[0, [0, 4, 8]] = 1.0 / _SQ3
_GS = np.repeat(np.eye(16, dtype=np.float32), 5, axis=0)


def _tc_body(n_total, xs_ref, e0_ref, e2a_ref, e2b_ref, w1_ref, b1_ref,
             wh2_ref, w2_ref, b2_ref, v_ref, gs_ref, gst_ref, mix_ref,
             out_ref):
    i = pl.program_id(0)
    f32 = jnp.float32
    bf16 = jnp.bfloat16
    a = jnp.concatenate([xs_ref[...], e0_ref[...]], axis=1).astype(bf16)
    h = jnp.dot(a, w1_ref[...], preferred_element_type=f32) + b1_ref[...]
    lane = lax.broadcasted_iota(jnp.int32, h.shape, 1)
    ag = h * jax.nn.sigmoid(jnp.where(lane < 64, h, jnp.abs(h)))
    p = jnp.dot(ag.astype(bf16), w2_ref[...],
                preferred_element_type=f32) + b2_ref[...]
    e2b = jnp.where(lane < 96, e2b_ref[...], 0.0)
    a2 = jnp.concatenate([e2a_ref[...], e2b], axis=1).astype(bf16)
    h2 = jnp.dot(a2, wh2_ref[...], preferred_element_type=f32)
    n2sq = jnp.dot((h2 * h2).astype(bf16), gs_ref[...],
                   preferred_element_type=f32)
    sig2 = jax.nn.sigmoid(jnp.sqrt(n2sq))
    g2 = h2 * jnp.dot(sig2.astype(bf16), gst_ref[...],
                      preferred_element_type=f32)
    s2 = jnp.dot(g2.astype(bf16), v_ref[...],
                 preferred_element_type=f32)
    zero = p[:, 2:3] * p[:, 0:1]
    second = s2 * p[:, 1:2]
    dn = jnp.sqrt(jnp.sum(second * second, axis=1, keepdims=True))
    feat = jnp.concatenate([zero, dn, second], axis=1)
    out16 = jnp.dot(feat.astype(bf16), mix_ref[...],
                    preferred_element_type=f32)
    rows = i * _T + lax.broadcasted_iota(jnp.int32, out16.shape, 0)
    out_ref[...] = jnp.where(rows < n_total, out16, 0.0)


def _mat_rows(x_scalar, x_spherical, w1, b1, wh2, w2, b2, v):
    n = x_scalar.shape[0]
    g = pl.cdiv(n, _T)
    npad = g * _T
    bf16 = jnp.bfloat16
    mix = jnp.asarray(np.concatenate([_ZR, _DR, _SM], axis=0), bf16)
    return pl.pallas_call(
        functools.partial(_tc_body, n),
        grid=(g,),
        in_specs=[
            pl.BlockSpec((_T, 128), lambda i: (i, 0)),
            pl.BlockSpec((_T, 128), lambda i: (i, 0)),
            pl.BlockSpec((_T, 128), lambda i: (i, 2)),
            pl.BlockSpec((_T, 128), lambda i: (i, 3)),
            pl.BlockSpec((256, 128), lambda i: (0, 0)),
            pl.BlockSpec((1, 128), lambda i: (0, 0)),
            pl.BlockSpec((256, 80), lambda i: (0, 0)),
            pl.BlockSpec((128, 8), lambda i: (0, 0)),
            pl.BlockSpec((1, 8), lambda i: (0, 0)),
            pl.BlockSpec((80, 5), lambda i: (0, 0)),
            pl.BlockSpec((80, 16), lambda i: (0, 0)),
            pl.BlockSpec((16, 80), lambda i: (0, 0)),
            pl.BlockSpec((7, 16), lambda i: (0, 0)),
        ],
        out_specs=pl.BlockSpec((_T, 16), lambda i: (i, 0)),
        out_shape=jax.ShapeDtypeStruct((npad, 16), jnp.float32),
    )(x_scalar, x_spherical, x_spherical, x_spherical, w1.astype(bf16), b1,
      wh2.astype(bf16), w2.astype(bf16), b2, v.astype(bf16),
      jnp.asarray(_GS, bf16), jnp.asarray(_GS.T, bf16), mix)


_SZ = 64
_SPW = 32


def _seg_sum(mats, idx2d, karr):
    mesh = plsc.VectorSubcoreMesh(core_axis_name="c", subcore_axis_name="s")

    @functools.partial(
        pl.kernel,
        out_type=jax.ShapeDtypeStruct((_NSEG, 16), jnp.float32),
        mesh=mesh,
        scratch_types=[
            pltpu.VMEM((16,), jnp.int32),
            pltpu.VMEM((_SZ,), jnp.int32),
            pltpu.VMEM((_SZ, 16), jnp.float32),
            pltpu.VMEM((_SPW + 1, 16), jnp.float32),
        ],
    )
    def sc_kernel(mat_hbm, idx_hbm, karr_hbm, out_hbm, kb, idx_v, buf, acc):
        c = lax.axis_index("c")
        s = lax.axis_index("s")
        wid = s * 2 + c
        base_seg = wid * _SPW
        for r in range(_SPW + 1):
            acc[r, :] = jnp.zeros((16,), jnp.float32)
        pltpu.sync_copy(karr_hbm.at[pl.ds(wid * 16, 16)], kb)
        kbv = kb[...]
        k0 = kbv[0]
        k1 = kbv[1]

        def step(k, carry):
            pltpu.sync_copy(idx_hbm.at[k], idx_v)
            pltpu.sync_copy(mat_hbm.at[pl.ds(k * _SZ, _SZ)], buf)
            for g in range(_SZ // 16):
                iv = idx_v[pl.ds(g * 16, 16)] - base_seg
                for r in range(16):
                    rel = iv[r]
                    ok = (rel >= 0) & (rel < _SPW)
                    ridx = jnp.where(ok, rel, _SPW)
                    plsc.addupdate(acc.at[ridx], buf[g * 16 + r, :])
            return carry

        lax.fori_loop(k0, k1, step, 0)
        pltpu.sync_copy(acc.at[pl.ds(0, _SPW)],
                        out_hbm.at[pl.ds(base_seg, _SPW)])

    return sc_kernel(mats, idx2d, karr)


def _prep_weights(sW1, sb1, sW2, sb2, lW0, lb0, lW2, vW0, vb0, vW2):
    f32 = jnp.float32
    w1 = jnp.zeros((256, 128), f32)
    w1 = w1.at[:128, :64].set(sW1)
    w1 = w1.at[128:, 64:].set(lW0 * (1.0 / math.sqrt(128.0)))
    b1 = jnp.concatenate([sb1, lb0]).reshape(1, 128)
    wh2_core = (lW2[:, None, :, None] * jnp.eye(5, dtype=f32)[None, :, None, :])
    wh2_core = wh2_core.reshape(160, 80) * (1.0 / math.sqrt(32.0))
    wh2 = jnp.zeros((256, 80), f32).at[64:224, :].set(wh2_core)
    w2 = jnp.zeros((128, 8), f32)
    w2 = w2.at[:64, 0:2].set(sW2)
    w2 = w2.at[64:, 2].set(vW0[:, 0] * (1.0 / math.sqrt(64.0)))
    b2 = jnp.zeros((8,), f32).at[0:2].set(sb2).at[2].set(vb0[0]).reshape(1, 8)
    v = (vW2[:, 0][:, None, None] * jnp.eye(5, dtype=f32)[None, :, :])
    v = v.reshape(80, 5) * (1.0 / math.sqrt(16.0))
    return w1, b1, wh2, w2, b2, v


def kernel(x_scalar, x_spherical, coord, batch_idx, sW1, sb1, sW2, sb2,
           lW0, lb0, lW1, lW2, vW0, vb0, vW2):
    del coord, lW1
    n = x_scalar.shape[0]
    w1, b1, wh2, w2, b2, v = _prep_weights(sW1, sb1, sW2, sb2, lW0, lb0,
                                           lW2, vW0, vb0, vW2)
    mats = _mat_rows(x_scalar, x_spherical, w1, b1, wh2, w2, b2, v)
    npad = mats.shape[0]
    bip = jnp.concatenate(
        [batch_idx,
         jnp.full((npad - n,), _NSEG - 1, jnp.int32)])
    starts = jnp.searchsorted(bip, jnp.arange(0, _NSEG + 1, _SPW)
                              ).astype(jnp.int32)
    k0 = starts[:-1] // _SZ
    k1 = (starts[1:] + _SZ - 1) // _SZ
    karr = jnp.zeros((32, 16), jnp.int32)
    karr = karr.at[:, 0].set(k0).at[:, 1].set(k1).reshape(-1)
    seg = _seg_sum(mats, bip.reshape(-1, _SZ), karr)
    return seg[:, :9].reshape(_NSEG, 3, 3)


# speedup vs baseline: 20.1014x; 1.3475x over previous
import functools
import math

import jax
import jax.numpy as jnp
import numpy as np
from jax import lax
from jax.experimental import pallas as pl
from jax.experimental.pallas import tpu as pltpu
from jax.experimental.pallas import tpu_sc as plsc

_NSEG = 1024
_T = 4096

_SQ3 = math.sqrt(3.0)

_SM = np.zeros((5, 16), np.float32)
_SM[0, 1] = _SM[0, 3] = 1.0
_SM[1, 5] = _SM[1, 7] = 1.0
_SM[2, 0] = _SM[2, 4] = -1.0 / _SQ3
_SM[2, 8] = 2.0 / _SQ3
_SM[3, 2] = _SM[3, 6] = 1.0
_SM[4, 0] = 1.0
_SM[4, 4] = -1.0
_ZR = np.zeros((1, 16), np.float32)
_ZR[0, [0, 4, 8]] = 1.0
_DR = np.zeros((1, 16), np.float32)
_DR[0, [0, 4, 8]] = 1.0 / _SQ3
_GS = np.repeat(np.eye(16, dtype=np.float32), 5, axis=0)


def _tc_body(n_total, xs_ref, e0_ref, e2a_ref, e2b_ref, w1_ref, b1_ref,
             wh2_ref, w2_ref, b2_ref, v_ref, vsm_ref, one5_ref, gs_ref,
             gst_ref, zr_ref, dr_ref, out_ref):
    i = pl.program_id(0)
    f32 = jnp.float32
    bf16 = jnp.bfloat16
    a = jnp.concatenate([xs_ref[...], e0_ref[...]], axis=1).astype(bf16)
    h = jnp.dot(a, w1_ref[...], preferred_element_type=f32) + b1_ref[...]
    lane = lax.broadcasted_iota(jnp.int32, h.shape, 1)
    ag = h * jax.nn.sigmoid(jnp.where(lane < 64, h, jnp.abs(h)))
    p = jnp.dot(ag.astype(bf16), w2_ref[...],
                preferred_element_type=f32) + b2_ref[...]
    e2b = jnp.where(lane < 96, e2b_ref[...], 0.0)
    a2 = jnp.concatenate([e2a_ref[...], e2b], axis=1).astype(bf16)
    h2 = jnp.dot(a2, wh2_ref[...], preferred_element_type=f32)
    n2sq = jnp.dot((h2 * h2).astype(bf16), gs_ref[...],
                   preferred_element_type=f32)
    sig2 = jax.nn.sigmoid(jnp.sqrt(n2sq))
    g2 = (h2 * jnp.dot(sig2.astype(bf16), gst_ref[...],
                       preferred_element_type=f32)).astype(bf16)
    s2 = jnp.dot(g2, v_ref[...], preferred_element_type=f32)
    q2 = jnp.dot(g2, vsm_ref[...], preferred_element_type=f32)
    nssq = jnp.dot((s2 * s2).astype(bf16), one5_ref[...],
                   preferred_element_type=f32)
    ns = jnp.sqrt(nssq)
    p0 = p[:, 0:1]
    p1 = p[:, 1:2]
    s0 = p[:, 2:3]
    out16 = (p1 * q2 + (p0 * s0) * zr_ref[...]
             + (jnp.abs(p1) * ns) * dr_ref[...])
    rows = i * _T + lax.broadcasted_iota(jnp.int32, out16.shape, 0)
    out_ref[...] = jnp.where(rows < n_total, out16, 0.0)


def _mat_rows(x_scalar, x_spherical, w1, b1, wh2, w2, b2, v):
    n = x_scalar.shape[0]
    g = pl.cdiv(n, _T)
    npad = g * _T
    bf16 = jnp.bfloat16
    return pl.pallas_call(
        functools.partial(_tc_body, n),
        grid=(g,),
        in_specs=[
            pl.BlockSpec((_T, 128), lambda i: (i, 0)),
            pl.BlockSpec((_T, 128), lambda i: (i, 0)),
            pl.BlockSpec((_T, 128), lambda i: (i, 2)),
            pl.BlockSpec((_T, 128), lambda i: (i, 3)),
            pl.BlockSpec((256, 128), lambda i: (0, 0)),
            pl.BlockSpec((1, 128), lambda i: (0, 0)),
            pl.BlockSpec((256, 80), lambda i: (0, 0)),
            pl.BlockSpec((128, 8), lambda i: (0, 0)),
            pl.BlockSpec((1, 8), lambda i: (0, 0)),
            pl.BlockSpec((80, 5), lambda i: (0, 0)),
            pl.BlockSpec((80, 16), lambda i: (0, 0)),
            pl.BlockSpec((5, 1), lambda i: (0, 0)),
            pl.BlockSpec((80, 16), lambda i: (0, 0)),
            pl.BlockSpec((16, 80), lambda i: (0, 0)),
            pl.BlockSpec((1, 16), lambda i: (0, 0)),
            pl.BlockSpec((1, 16), lambda i: (0, 0)),
        ],
        out_specs=pl.BlockSpec((_T, 16), lambda i: (i, 0)),
        out_shape=jax.ShapeDtypeStruct((npad, 16), jnp.float32),
    )(x_scalar, x_spherical, x_spherical, x_spherical, w1.astype(bf16), b1,
      wh2.astype(bf16), w2.astype(bf16), b2, v.astype(bf16),
      (v @ jnp.asarray(_SM)).astype(bf16), jnp.ones((5, 1), bf16),
      jnp.asarray(_GS, bf16), jnp.asarray(_GS.T, bf16),
      jnp.asarray(_ZR), jnp.asarray(_DR))


_SZ = 64
_SPW = 32


def _seg_sum(mats, idx2d, karr):
    mesh = plsc.VectorSubcoreMesh(core_axis_name="c", subcore_axis_name="s")

    @functools.partial(
        pl.kernel,
        out_type=jax.ShapeDtypeStruct((_NSEG, 16), jnp.float32),
        mesh=mesh,
        scratch_types=[
            pltpu.VMEM((16,), jnp.int32),
            pltpu.VMEM((2, _SZ), jnp.int32),
            pltpu.VMEM((2, _SZ, 16), jnp.float32),
            pltpu.VMEM((_SPW + 1, 16), jnp.float32),
            pltpu.SemaphoreType.DMA,
            pltpu.SemaphoreType.DMA,
            pltpu.SemaphoreType.DMA,
            pltpu.SemaphoreType.DMA,
        ],
    )
    def sc_kernel(mat_hbm, idx_hbm, karr_hbm, out_hbm, kb, idx_v, buf, acc,
                  si0, si1, sb0, sb1):
        c = lax.axis_index("c")
        s = lax.axis_index("s")
        wid = s * 2 + c
        base_seg = wid * _SPW
        for r in range(_SPW + 1):
            acc[r, :] = jnp.zeros((16,), jnp.float32)
        pltpu.sync_copy(karr_hbm.at[pl.ds(wid * 16, 16)], kb)
        kbv = kb[...]
        k0 = kbv[0]
        k1 = kbv[1]
        sis = (si0, si1)
        sbs = (sb0, sb1)

        def fire(k, slot):
            pltpu.async_copy(idx_hbm.at[k], idx_v.at[slot], sis[slot])
            pltpu.async_copy(mat_hbm.at[pl.ds(k * _SZ, _SZ)], buf.at[slot],
                             sbs[slot])

        @pl.when(k1 > k0)
        def _():
            fire(k0, 0)

        @pl.when(k1 > k0 + 1)
        def _():
            fire(k0 + 1, 1)

        def step(i, carry):
            for slot in range(2):
                k = k0 + 2 * i + slot

                @pl.when(k < k1)
                def _():
                    pltpu.make_async_copy(idx_hbm.at[k], idx_v.at[slot],
                                          sis[slot]).wait()
                    pltpu.make_async_copy(mat_hbm.at[pl.ds(k * _SZ, _SZ)],
                                          buf.at[slot], sbs[slot]).wait()

                    @pl.when(k + 2 < k1)
                    def _():
                        fire(k + 2, slot)

                    for g in range(_SZ // 16):
                        iv = idx_v[slot, pl.ds(g * 16, 16)] - base_seg
                        for r in range(16):
                            rel = iv[r]
                            ok = (rel >= 0) & (rel < _SPW)
                            ridx = jnp.where(ok, rel, _SPW)
                            plsc.addupdate(acc.at[ridx], buf[slot, g * 16 + r, :])
            return carry

        lax.fori_loop(0, (k1 - k0 + 1) // 2, step, 0)
        pltpu.sync_copy(acc.at[pl.ds(0, _SPW)],
                        out_hbm.at[pl.ds(base_seg, _SPW)])

    return sc_kernel(mats, idx2d, karr)


def _prep_weights(sW1, sb1, sW2, sb2, lW0, lb0, lW2, vW0, vb0, vW2):
    f32 = jnp.float32
    w1 = jnp.zeros((256, 128), f32)
    w1 = w1.at[:128, :64].set(sW1)
    w1 = w1.at[128:, 64:].set(lW0 * (1.0 / math.sqrt(128.0)))
    b1 = jnp.concatenate([sb1, lb0]).reshape(1, 128)
    wh2_core = (lW2[:, None, :, None] * jnp.eye(5, dtype=f32)[None, :, None, :])
    wh2_core = wh2_core.reshape(160, 80) * (1.0 / math.sqrt(32.0))
    wh2 = jnp.zeros((256, 80), f32).at[64:224, :].set(wh2_core)
    w2 = jnp.zeros((128, 8), f32)
    w2 = w2.at[:64, 0:2].set(sW2)
    w2 = w2.at[64:, 2].set(vW0[:, 0] * (1.0 / math.sqrt(64.0)))
    b2 = jnp.zeros((8,), f32).at[0:2].set(sb2).at[2].set(vb0[0]).reshape(1, 8)
    v = (vW2[:, 0][:, None, None] * jnp.eye(5, dtype=f32)[None, :, :])
    v = v.reshape(80, 5) * (1.0 / math.sqrt(16.0))
    return w1, b1, wh2, w2, b2, v


def kernel(x_scalar, x_spherical, coord, batch_idx, sW1, sb1, sW2, sb2,
           lW0, lb0, lW1, lW2, vW0, vb0, vW2):
    del coord, lW1
    n = x_scalar.shape[0]
    w1, b1, wh2, w2, b2, v = _prep_weights(sW1, sb1, sW2, sb2, lW0, lb0,
                                           lW2, vW0, vb0, vW2)
    mats = _mat_rows(x_scalar, x_spherical, w1, b1, wh2, w2, b2, v)
    npad = mats.shape[0]
    bip = jnp.concatenate(
        [batch_idx,
         jnp.full((npad - n,), _NSEG - 1, jnp.int32)])
    starts = jnp.searchsorted(bip, jnp.arange(0, _NSEG + 1, _SPW)
                              ).astype(jnp.int32)
    k0 = starts[:-1] // _SZ
    k1 = (starts[1:] + _SZ - 1) // _SZ
    karr = jnp.zeros((32, 16), jnp.int32)
    karr = karr.at[:, 0].set(k0).at[:, 1].set(k1).reshape(-1)
    seg = _seg_sum(mats, bip.reshape(-1, _SZ), karr)
    return seg[:, :9].reshape(_NSEG, 3, 3)
